# baseline (speedup 1.0000x reference)
import functools

import jax
import jax.numpy as jnp
from jax import lax
from jax.experimental import pallas as pl
from jax.experimental.pallas import tpu as pltpu
from jax.experimental.pallas import tpu_sc as plsc

NC = 2
NS = 16
L = 16
G = 4


def _probs_kernel(x_ref, wt_ref, probs_ref, imp_ref, imp_acc, *, bt):
    i = pl.program_id(0)
    logits = jnp.dot(x_ref[...], wt_ref[...],
                     preferred_element_type=jnp.float32)
    m = jnp.max(logits, axis=-1, keepdims=True)
    e = jnp.exp(logits - m)
    s = jnp.sum(e, axis=-1, keepdims=True)
    probs = e / s
    probs_ref[...] = probs

    imp_part = jnp.sum(probs, axis=0, keepdims=True)

    @pl.when(i == 0)
    def _():
        imp_acc[...] = imp_part

    @pl.when(i > 0)
    def _():
        imp_acc[...] += imp_part

    @pl.when(i == pl.num_programs(0) - 1)
    def _():
        imp_ref[...] = imp_acc[...]


def _tc_probs(x, wt):
    n, d = x.shape
    ne = wt.shape[1]
    bt = 1024
    return pl.pallas_call(
        functools.partial(_probs_kernel, bt=bt),
        grid=(n // bt,),
        in_specs=[
            pl.BlockSpec((bt, d), lambda i: (i, 0)),
            pl.BlockSpec((d, ne), lambda i: (0, 0)),
        ],
        out_specs=[
            pl.BlockSpec((bt, ne), lambda i: (i, 0)),
            pl.BlockSpec((1, ne), lambda i: (0, 0)),
        ],
        out_shape=[
            jax.ShapeDtypeStruct((n, ne), jnp.float32),
            jax.ShapeDtypeStruct((1, ne), jnp.float32),
        ],
        scratch_shapes=[pltpu.VMEM((1, ne), jnp.float32)],
    )(x, wt)


def _route_body(probs_hbm, tp_hbm, ti_hbm,
                probs_v, tp_v, ti_v, *, tpw, ne):
    wid = lax.axis_index("s") * NC + lax.axis_index("c")
    base = wid * tpw
    pltpu.sync_copy(probs_hbm.at[pl.ds(base * ne, tpw * ne)], probs_v)

    lane = lax.broadcasted_iota(jnp.int32, (L,), 0)
    ngroups = tpw // L

    def block_body(b, _):
        rows = [lane + (b * G + gg) * L for gg in range(G)]
        flat = [r * ne for r in rows]
        p1 = [jnp.full((L,), -1.0, jnp.float32) for _ in range(G)]
        i1 = [jnp.zeros((L,), jnp.int32) for _ in range(G)]
        p2 = [jnp.full((L,), -1.0, jnp.float32) for _ in range(G)]
        i2 = [jnp.zeros((L,), jnp.int32) for _ in range(G)]

        for e in range(ne):
            colv = jnp.full((L,), e, jnp.int32)
            for gg in range(G):
                v = plsc.load_gather(probs_v, [flat[gg] + e])
                gt1 = v > p1[gg]
                gt2 = v > p2[gg]
                p2[gg] = jnp.where(gt1, p1[gg], jnp.where(gt2, v, p2[gg]))
                i2[gg] = jnp.where(gt1, i1[gg], jnp.where(gt2, colv, i2[gg]))
                p1[gg] = jnp.where(gt1, v, p1[gg])
                i1[gg] = jnp.where(gt1, colv, i1[gg])

        for gg in range(G):
            denom = p1[gg] + p2[gg]
            plsc.store_scatter(tp_v, [rows[gg] * 2], p1[gg] / denom)
            plsc.store_scatter(tp_v, [rows[gg] * 2 + 1], p2[gg] / denom)
            plsc.store_scatter(ti_v, [rows[gg] * 2], i1[gg])
            plsc.store_scatter(ti_v, [rows[gg] * 2 + 1], i2[gg])
        return 0

    lax.fori_loop(0, ngroups // G, block_body, 0)

    pltpu.sync_copy(tp_v, tp_hbm.at[pl.ds(base * 2, tpw * 2)])
    pltpu.sync_copy(ti_v, ti_hbm.at[pl.ds(base * 2, tpw * 2)])


def _sc_route(probs):
    n, ne = probs.shape
    probs_flat = probs.reshape(n * ne)
    nw = NC * NS
    tpw = n // nw
    mesh = plsc.VectorSubcoreMesh(core_axis_name="c", subcore_axis_name="s",
                                  num_cores=NC, num_subcores=NS)
    tp, ti = pl.kernel(
        functools.partial(_route_body, tpw=tpw, ne=ne),
        out_type=[
            jax.ShapeDtypeStruct((n * 2,), jnp.float32),
            jax.ShapeDtypeStruct((n * 2,), jnp.int32),
        ],
        mesh=mesh,
        scratch_types=[
            pltpu.VMEM((tpw * ne,), jnp.float32),
            pltpu.VMEM((tpw * 2,), jnp.float32),
            pltpu.VMEM((tpw * 2,), jnp.int32),
        ],
        compiler_params=pltpu.CompilerParams(needs_layout_passes=False),
    )(probs_flat)
    return tp, ti


def _mask_aux_kernel(imp_ref, ti_ref, mask_ref, aux_ref, load_acc,
                     *, bt, ne, n_tokens):
    i = pl.program_id(0)
    ti = ti_ref[...]
    col = lax.broadcasted_iota(jnp.int32, (bt, ne), 1)
    mask = ((col == ti[:, 0:1]) | (col == ti[:, 1:2])).astype(jnp.float32)
    mask_ref[...] = mask
    load_part = jnp.sum(mask, axis=0, keepdims=True)

    @pl.when(i == 0)
    def _():
        load_acc[...] = load_part

    @pl.when(i > 0)
    def _():
        load_acc[...] += load_part

    @pl.when(i == pl.num_programs(0) - 1)
    def _():
        scale = ne / (n_tokens * n_tokens + 1e-06)
        aux_ref[...] = jnp.sum(imp_ref[...] * load_acc[...],
                               keepdims=True).reshape(1, 1) * scale


def _tc_mask_aux(imp, ti2d, n, ne):
    bt = 2048
    return pl.pallas_call(
        functools.partial(_mask_aux_kernel, bt=bt, ne=ne, n_tokens=n),
        grid=(n // bt,),
        in_specs=[
            pl.BlockSpec((1, ne), lambda i: (0, 0)),
            pl.BlockSpec((bt, 2), lambda i: (i, 0)),
        ],
        out_specs=[
            pl.BlockSpec((bt, ne), lambda i: (i, 0)),
            pl.BlockSpec((1, 1), lambda i: (0, 0)),
        ],
        out_shape=[
            jax.ShapeDtypeStruct((n, ne), jnp.float32),
            jax.ShapeDtypeStruct((1, 1), jnp.float32),
        ],
        scratch_shapes=[pltpu.VMEM((1, ne), jnp.float32)],
    )(imp, ti2d)


def kernel(x, W):
    n = x.shape[0]
    ne = W.shape[0]
    probs, imp = _tc_probs(x, W.T)
    tp, ti = _sc_route(probs)
    ti2d = ti.reshape(n, 2)
    mask, aux = _tc_mask_aux(imp, ti2d, n, ne)
    return tp.reshape(n, 2), ti2d, aux[0, 0], mask


# speedup vs baseline: 1.0356x; 1.0356x over previous
import functools

import jax
import jax.numpy as jnp
from jax import lax
from jax.experimental import pallas as pl
from jax.experimental.pallas import tpu as pltpu
from jax.experimental.pallas import tpu_sc as plsc

NC = 2
NS = 16
L = 16
G = 4


def _probs_kernel(x_ref, wt_ref, probs_ref, imp_ref, imp_acc, *, bt):
    i = pl.program_id(0)
    logits = jnp.dot(x_ref[...], wt_ref[...],
                     preferred_element_type=jnp.float32)
    m = jnp.max(logits, axis=-1, keepdims=True)
    e = jnp.exp(logits - m)
    s = jnp.sum(e, axis=-1, keepdims=True)
    probs = e / s
    probs_ref[...] = probs

    imp_part = jnp.sum(probs, axis=0, keepdims=True)

    @pl.when(i == 0)
    def _():
        imp_acc[...] = imp_part

    @pl.when(i > 0)
    def _():
        imp_acc[...] += imp_part

    @pl.when(i == pl.num_programs(0) - 1)
    def _():
        imp_ref[...] = imp_acc[...]


def _tc_probs(x, wt):
    n, d = x.shape
    ne = wt.shape[1]
    bt = 1024
    return pl.pallas_call(
        functools.partial(_probs_kernel, bt=bt),
        grid=(n // bt,),
        in_specs=[
            pl.BlockSpec((bt, d), lambda i: (i, 0)),
            pl.BlockSpec((d, ne), lambda i: (0, 0)),
        ],
        out_specs=[
            pl.BlockSpec((bt, ne), lambda i: (i, 0)),
            pl.BlockSpec((1, ne), lambda i: (0, 0)),
        ],
        out_shape=[
            jax.ShapeDtypeStruct((n, ne), jnp.float32),
            jax.ShapeDtypeStruct((1, ne), jnp.float32),
        ],
        scratch_shapes=[pltpu.VMEM((1, ne), jnp.float32)],
    )(x, wt)


def _route_body(probs_hbm, tp_hbm, ti_hbm,
                probs_v, tp_v, ti_v, *, tpw, ne):
    wid = lax.axis_index("s") * NC + lax.axis_index("c")
    base = wid * tpw
    pltpu.sync_copy(probs_hbm.at[pl.ds(base, tpw)], probs_v)

    lane = lax.broadcasted_iota(jnp.int32, (L,), 0)
    ngroups = tpw // L

    def block_body(b, _):
        rows = [lane + (b * G + gg) * L for gg in range(G)]
        p1 = [jnp.full((L,), -1.0, jnp.float32) for _ in range(G)]
        i1 = [jnp.zeros((L,), jnp.int32) for _ in range(G)]
        p2 = [jnp.full((L,), -1.0, jnp.float32) for _ in range(G)]
        i2 = [jnp.zeros((L,), jnp.int32) for _ in range(G)]

        for e in range(ne):
            colv = jnp.full((L,), e, jnp.int32)
            for gg in range(G):
                v = plsc.load_gather(probs_v, [rows[gg], colv])
                gt1 = v > p1[gg]
                gt2 = v > p2[gg]
                p2[gg] = jnp.where(gt1, p1[gg], jnp.where(gt2, v, p2[gg]))
                i2[gg] = jnp.where(gt1, i1[gg], jnp.where(gt2, colv, i2[gg]))
                p1[gg] = jnp.where(gt1, v, p1[gg])
                i1[gg] = jnp.where(gt1, colv, i1[gg])

        for gg in range(G):
            denom = p1[gg] + p2[gg]
            plsc.store_scatter(tp_v, [rows[gg] * 2], p1[gg] / denom)
            plsc.store_scatter(tp_v, [rows[gg] * 2 + 1], p2[gg] / denom)
            plsc.store_scatter(ti_v, [rows[gg] * 2], i1[gg])
            plsc.store_scatter(ti_v, [rows[gg] * 2 + 1], i2[gg])
        return 0

    lax.fori_loop(0, ngroups // G, block_body, 0)

    pltpu.sync_copy(tp_v, tp_hbm.at[pl.ds(base * 2, tpw * 2)])
    pltpu.sync_copy(ti_v, ti_hbm.at[pl.ds(base * 2, tpw * 2)])


def _sc_route(probs):
    n, ne = probs.shape
    nw = NC * NS
    tpw = n // nw
    mesh = plsc.VectorSubcoreMesh(core_axis_name="c", subcore_axis_name="s",
                                  num_cores=NC, num_subcores=NS)
    tp, ti = pl.kernel(
        functools.partial(_route_body, tpw=tpw, ne=ne),
        out_type=[
            jax.ShapeDtypeStruct((n * 2,), jnp.float32),
            jax.ShapeDtypeStruct((n * 2,), jnp.int32),
        ],
        mesh=mesh,
        scratch_types=[
            pltpu.VMEM((tpw, ne), jnp.float32),
            pltpu.VMEM((tpw * 2,), jnp.float32),
            pltpu.VMEM((tpw * 2,), jnp.int32),
        ],
        compiler_params=pltpu.CompilerParams(needs_layout_passes=False),
    )(probs)
    return tp, ti


def _mask_aux_kernel(imp_ref, ti_ref, mask_ref, aux_ref, load_acc,
                     *, bt, ne, n_tokens):
    i = pl.program_id(0)
    ti = ti_ref[...]
    col = lax.broadcasted_iota(jnp.int32, (bt, ne), 1)
    mask = ((col == ti[:, 0:1]) | (col == ti[:, 1:2])).astype(jnp.float32)
    mask_ref[...] = mask
    load_part = jnp.sum(mask, axis=0, keepdims=True)

    @pl.when(i == 0)
    def _():
        load_acc[...] = load_part

    @pl.when(i > 0)
    def _():
        load_acc[...] += load_part

    @pl.when(i == pl.num_programs(0) - 1)
    def _():
        scale = ne / (n_tokens * n_tokens + 1e-06)
        aux_ref[...] = jnp.sum(imp_ref[...] * load_acc[...],
                               keepdims=True).reshape(1, 1) * scale


def _tc_mask_aux(imp, ti2d, n, ne):
    bt = 2048
    return pl.pallas_call(
        functools.partial(_mask_aux_kernel, bt=bt, ne=ne, n_tokens=n),
        grid=(n // bt,),
        in_specs=[
            pl.BlockSpec((1, ne), lambda i: (0, 0)),
            pl.BlockSpec((bt, 2), lambda i: (i, 0)),
        ],
        out_specs=[
            pl.BlockSpec((bt, ne), lambda i: (i, 0)),
            pl.BlockSpec((1, 1), lambda i: (0, 0)),
        ],
        out_shape=[
            jax.ShapeDtypeStruct((n, ne), jnp.float32),
            jax.ShapeDtypeStruct((1, 1), jnp.float32),
        ],
        scratch_shapes=[pltpu.VMEM((1, ne), jnp.float32)],
    )(imp, ti2d)


def kernel(x, W):
    n = x.shape[0]
    ne = W.shape[0]
    probs, imp = _tc_probs(x, W.T)
    tp, ti = _sc_route(probs)
    ti2d = ti.reshape(n, 2)
    mask, aux = _tc_mask_aux(imp, ti2d, n, ne)
    return tp.reshape(n, 2), ti2d, aux[0, 0], mask


# speedup vs baseline: 1.0848x; 1.0476x over previous
import functools

import jax
import jax.numpy as jnp
from jax import lax
from jax.experimental import pallas as pl
from jax.experimental.pallas import tpu as pltpu
from jax.experimental.pallas import tpu_sc as plsc

NC = 2
NS = 16
L = 16
G = 4


def _probs_kernel(x_ref, wt_ref, probs_ref, imp_ref, imp_acc, *, bt):
    i = pl.program_id(0)
    logits = jnp.dot(x_ref[...], wt_ref[...],
                     preferred_element_type=jnp.float32)
    m = jnp.max(logits, axis=-1, keepdims=True)
    e = jnp.exp(logits - m)
    s = jnp.sum(e, axis=-1, keepdims=True)
    probs = e / s
    probs_ref[...] = probs

    imp_part = jnp.sum(probs, axis=0, keepdims=True)

    @pl.when(i == 0)
    def _():
        imp_acc[...] = imp_part

    @pl.when(i > 0)
    def _():
        imp_acc[...] += imp_part

    @pl.when(i == pl.num_programs(0) - 1)
    def _():
        imp_ref[...] = imp_acc[...]


def _tc_probs(x, wt, c, nchunks):
    n, d = x.shape
    ne = wt.shape[1]
    bt = 1024
    nc_tok = n // nchunks
    steps = nc_tok // bt
    off = c * steps
    return pl.pallas_call(
        functools.partial(_probs_kernel, bt=bt),
        grid=(steps,),
        in_specs=[
            pl.BlockSpec((bt, d), lambda i: (i + off, 0)),
            pl.BlockSpec((d, ne), lambda i: (0, 0)),
        ],
        out_specs=[
            pl.BlockSpec((bt, ne), lambda i: (i, 0)),
            pl.BlockSpec((1, ne), lambda i: (0, 0)),
        ],
        out_shape=[
            jax.ShapeDtypeStruct((nc_tok, ne), jnp.float32),
            jax.ShapeDtypeStruct((1, ne), jnp.float32),
        ],
        scratch_shapes=[pltpu.VMEM((1, ne), jnp.float32)],
    )(x, wt)


def _route_body(probs_hbm, tp_hbm, ti_hbm,
                probs_v, tp_v, ti_v, *, tpw, ne):
    wid = lax.axis_index("s") * NC + lax.axis_index("c")
    base = wid * tpw
    pltpu.sync_copy(probs_hbm.at[pl.ds(base, tpw)], probs_v)

    lane = lax.broadcasted_iota(jnp.int32, (L,), 0)
    ngroups = tpw // L

    def block_body(b, _):
        rows = [lane + (b * G + gg) * L for gg in range(G)]
        p1 = [jnp.full((L,), -1.0, jnp.float32) for _ in range(G)]
        i1 = [jnp.zeros((L,), jnp.int32) for _ in range(G)]
        p2 = [jnp.full((L,), -1.0, jnp.float32) for _ in range(G)]
        i2 = [jnp.zeros((L,), jnp.int32) for _ in range(G)]

        for e in range(ne):
            colv = jnp.full((L,), e, jnp.int32)
            for gg in range(G):
                v = plsc.load_gather(probs_v, [rows[gg], colv])
                gt1 = v > p1[gg]
                gt2 = v > p2[gg]
                p2[gg] = jnp.where(gt1, p1[gg], jnp.where(gt2, v, p2[gg]))
                i2[gg] = jnp.where(gt1, i1[gg], jnp.where(gt2, colv, i2[gg]))
                p1[gg] = jnp.where(gt1, v, p1[gg])
                i1[gg] = jnp.where(gt1, colv, i1[gg])

        for gg in range(G):
            denom = p1[gg] + p2[gg]
            plsc.store_scatter(tp_v, [rows[gg] * 2], p1[gg] / denom)
            plsc.store_scatter(tp_v, [rows[gg] * 2 + 1], p2[gg] / denom)
            plsc.store_scatter(ti_v, [rows[gg] * 2], i1[gg])
            plsc.store_scatter(ti_v, [rows[gg] * 2 + 1], i2[gg])
        return 0

    lax.fori_loop(0, ngroups // G, block_body, 0)

    pltpu.sync_copy(tp_v, tp_hbm.at[pl.ds(base * 2, tpw * 2)])
    pltpu.sync_copy(ti_v, ti_hbm.at[pl.ds(base * 2, tpw * 2)])


def _sc_route(probs):
    n, ne = probs.shape
    nw = NC * NS
    tpw = n // nw
    mesh = plsc.VectorSubcoreMesh(core_axis_name="c", subcore_axis_name="s",
                                  num_cores=NC, num_subcores=NS)
    tp, ti = pl.kernel(
        functools.partial(_route_body, tpw=tpw, ne=ne),
        out_type=[
            jax.ShapeDtypeStruct((n * 2,), jnp.float32),
            jax.ShapeDtypeStruct((n * 2,), jnp.int32),
        ],
        mesh=mesh,
        scratch_types=[
            pltpu.VMEM((tpw, ne), jnp.float32),
            pltpu.VMEM((tpw * 2,), jnp.float32),
            pltpu.VMEM((tpw * 2,), jnp.int32),
        ],
        compiler_params=pltpu.CompilerParams(needs_layout_passes=False),
    )(probs)
    return tp, ti


def _mask_aux_kernel(imp0_ref, imp1_ref, ti_ref, mask_ref, aux_ref,
                     load_acc, *, bt, ne, n_tokens):
    i = pl.program_id(0)
    ti = ti_ref[...]
    col = lax.broadcasted_iota(jnp.int32, (bt, ne), 1)
    mask = ((col == ti[:, 0:1]) | (col == ti[:, 1:2])).astype(jnp.float32)
    mask_ref[...] = mask
    load_part = jnp.sum(mask, axis=0, keepdims=True)

    @pl.when(i == 0)
    def _():
        load_acc[...] = load_part

    @pl.when(i > 0)
    def _():
        load_acc[...] += load_part

    @pl.when(i == pl.num_programs(0) - 1)
    def _():
        scale = ne / (n_tokens * n_tokens + 1e-06)
        imp = imp0_ref[...] + imp1_ref[...]
        aux_ref[...] = jnp.sum(imp * load_acc[...],
                               keepdims=True).reshape(1, 1) * scale


def _tc_mask_aux(imp0, imp1, ti2d, n, ne):
    bt = 2048
    return pl.pallas_call(
        functools.partial(_mask_aux_kernel, bt=bt, ne=ne, n_tokens=n),
        grid=(n // bt,),
        in_specs=[
            pl.BlockSpec((1, ne), lambda i: (0, 0)),
            pl.BlockSpec((1, ne), lambda i: (0, 0)),
            pl.BlockSpec((bt, 2), lambda i: (i, 0)),
        ],
        out_specs=[
            pl.BlockSpec((bt, ne), lambda i: (i, 0)),
            pl.BlockSpec((1, 1), lambda i: (0, 0)),
        ],
        out_shape=[
            jax.ShapeDtypeStruct((n, ne), jnp.float32),
            jax.ShapeDtypeStruct((1, 1), jnp.float32),
        ],
        scratch_shapes=[pltpu.VMEM((1, ne), jnp.float32)],
    )(imp0, imp1, ti2d)


def kernel(x, W):
    n = x.shape[0]
    ne = W.shape[0]
    wt = W.T
    nchunks = 2
    nc_tok = n // nchunks
    probs0, imp0 = _tc_probs(x, wt, 0, nchunks)
    tp0, ti0 = _sc_route(probs0)
    probs1, imp1 = _tc_probs(x, wt, 1, nchunks)
    tp1, ti1 = _sc_route(probs1)
    tp = jnp.concatenate([tp0.reshape(nc_tok, 2), tp1.reshape(nc_tok, 2)])
    ti2d = jnp.concatenate([ti0.reshape(nc_tok, 2), ti1.reshape(nc_tok, 2)])
    mask, aux = _tc_mask_aux(imp0, imp1, ti2d, n, ne)
    return tp, ti2d, aux[0, 0], mask
